# baseline (speedup 1.0000x reference)
import jax
import jax.numpy as jnp
from jax.experimental import pallas as pl
from jax.experimental.pallas import tpu as pltpu

_BM = 512


def _body(a0, a1, emb_ref, o0, o1):
    o0[...] = jnp.dot(a0[...], emb_ref[...], preferred_element_type=jnp.float32)
    o1[...] = jnp.dot(a1[...], emb_ref[...], preferred_element_type=jnp.float32)


def kernel(adj, embeds):
    M, K = adj.shape
    _, N = embeds.shape
    half_blocks = M // _BM // 2
    out = pl.pallas_call(
        _body,
        grid=(half_blocks,),
        in_specs=[
            pl.BlockSpec((_BM, K), lambda i: (i, 0)),
            pl.BlockSpec((_BM, K), lambda i: (i + 4, 0)),
            pl.BlockSpec((K, N), lambda i: (0, 0)),
        ],
        out_specs=[
            pl.BlockSpec((_BM, N), lambda i: (i, 0)),
            pl.BlockSpec((_BM, N), lambda i: (i, 0)),
        ],
        out_shape=[
            jax.ShapeDtypeStruct((M // 2, N), jnp.float32),
            jax.ShapeDtypeStruct((M // 2, N), jnp.float32),
        ],
        compiler_params=pltpu.CompilerParams(
            dimension_semantics=("arbitrary",),
        ),
    )(adj, adj, embeds)
    return jnp.concatenate(out, axis=0)


# speedup vs baseline: 1.0606x; 1.0606x over previous
import jax
import jax.numpy as jnp
from jax.experimental import pallas as pl
from jax.experimental.pallas import tpu as pltpu

_BM = 512


def _body(adj_ref, emb_ref, out_ref):
    out_ref[...] = jnp.dot(adj_ref[...], emb_ref[...],
                           preferred_element_type=jnp.float32)


def kernel(adj, embeds):
    M, K = adj.shape
    _, N = embeds.shape
    return pl.pallas_call(
        _body,
        grid=(M // _BM,),
        in_specs=[
            pl.BlockSpec((_BM, K), lambda i: (i, 0)),
            pl.BlockSpec((K, N), lambda i: (0, 0)),
        ],
        out_specs=pl.BlockSpec((_BM, N), lambda i: (i, 0)),
        out_shape=jax.ShapeDtypeStruct((M, N), jnp.float32),
        compiler_params=pltpu.CompilerParams(
            dimension_semantics=("parallel",),
        ),
    )(adj, embeds)
